# Initial kernel scaffold; baseline (speedup 1.0000x reference)
import jax
import jax.numpy as jnp
from jax.experimental import pallas as pl


def kernel(instance_embs_batch, edge_index, instance_batch_local_token_ids, node_counts, W1, al1, ar1, b1, W2, al2, ar2, b2):
    raise NotImplementedError("write your pallas kernel here")


# speedup vs baseline: 1.1124x; 1.1124x over previous
import functools

import jax
import jax.numpy as jnp
from jax.experimental import pallas as pl


def _mm_body(x_ref, w_ref, o_ref):
    o_ref[...] = jnp.dot(x_ref[...], w_ref[...], preferred_element_type=jnp.float32)


def _matmul(x, w, bn=1000):
    n, k = x.shape
    m = w.shape[1]
    return pl.pallas_call(
        _mm_body,
        grid=(n // bn,),
        in_specs=[pl.BlockSpec((bn, k), lambda i: (i, 0)),
                  pl.BlockSpec((k, m), lambda i: (0, 0))],
        out_specs=pl.BlockSpec((bn, m), lambda i: (i, 0)),
        out_shape=jax.ShapeDtypeStruct((n, m), jnp.float32),
    )(x, w)


def _gat_layer(x, src, dst, W, al, ar, b, heads, out_dim):
    n = x.shape[0]
    feat = _matmul(x, W).reshape(n, heads, out_dim)
    el = (feat * al[None, :, :]).sum(-1)
    er = (feat * ar[None, :, :]).sum(-1)
    e = el[src] + er[dst]
    e = jnp.where(e > 0, e, 0.2 * e)
    ee = jnp.exp(e)
    denom = jax.ops.segment_sum(ee, dst, num_segments=n)
    num = jax.ops.segment_sum(feat[src] * ee[..., None], dst, num_segments=n)
    rst = jnp.where(denom[..., None] > 0, num / jnp.where(denom[..., None] > 0, denom[..., None], 1.0), 0.0)
    return rst + b[None, :, :]


def kernel(instance_embs_batch, edge_index, instance_batch_local_token_ids, node_counts, W1, al1, ar1, b1, W2, al2, ar2, b2):
    src = edge_index[0]
    dst = edge_index[1]
    n = instance_embs_batch.shape[0]
    heads, hid = al1.shape
    out_dim = al2.shape[1]

    h = _gat_layer(instance_embs_batch, src, dst, W1, al1, ar1, b1, heads, hid)
    h = jax.nn.relu(h).reshape(n, heads * hid)
    h = _gat_layer(h, src, dst, W2, al2, ar2, b2, 1, out_dim)
    h = h.reshape(n, out_dim)

    offsets = jnp.concatenate([jnp.zeros((1,), dtype=node_counts.dtype), jnp.cumsum(node_counts)[:-1]])
    gid = instance_batch_local_token_ids + offsets[:, None]
    return jnp.squeeze(h[gid])


# speedup vs baseline: 4.9149x; 4.4182x over previous
import functools

import jax
import jax.numpy as jnp
from jax import lax
from jax.experimental import pallas as pl
from jax.experimental.pallas import tpu as pltpu
from jax.experimental.pallas import tpu_sc as plsc


def _xin(x):
    if x.ndim == 2:
        rows, k = x.shape
        return rows, k, x, lambda bn: pl.BlockSpec((bn, k), lambda i: (i, 0)), \
            lambda r: r[...]
    kc, rows, _ = x.shape
    return rows, kc * 128, x, \
        lambda bn: pl.BlockSpec((kc, bn, 128), lambda i: (0, i, 0)), \
        lambda r: jnp.concatenate([r[i] for i in range(kc)], axis=-1)


def _featT_body(nchunk, get, x_ref, w_ref, o_ref):
    f = jnp.dot(get(x_ref), w_ref[...], preferred_element_type=jnp.float32)
    for c in range(nchunk):
        o_ref[c] = f[:, c * 128:(c + 1) * 128]


def _featT(x, w, bn=1000):
    rows, k, x, xspec, get = _xin(x)
    m = w.shape[1]
    nchunk = m // 128
    return pl.pallas_call(
        functools.partial(_featT_body, nchunk, get),
        grid=(rows // bn,),
        in_specs=[xspec(bn), pl.BlockSpec((k, m), lambda i: (0, 0))],
        out_specs=pl.BlockSpec((nchunk, bn, 128), lambda i: (0, i, 0)),
        out_shape=jax.ShapeDtypeStruct((nchunk, rows, 128), jnp.float32),
    )(x, w)


def _elr_body(get, x_ref, pl_ref, pr_ref, el_ref, er_ref):
    xv = get(x_ref)
    el_ref[...] = jnp.dot(xv, pl_ref[...], preferred_element_type=jnp.float32)
    er_ref[...] = jnp.dot(xv, pr_ref[...], preferred_element_type=jnp.float32)


def _elr(x, pmat_l, pmat_r, bn=1000):
    rows, k, x, xspec, get = _xin(x)
    return pl.pallas_call(
        functools.partial(_elr_body, get),
        grid=(rows // bn,),
        in_specs=[xspec(bn),
                  pl.BlockSpec((k, 128), lambda i: (0, 0)),
                  pl.BlockSpec((k, 128), lambda i: (0, 0))],
        out_specs=[pl.BlockSpec((bn, 128), lambda i: (i, 0)),
                   pl.BlockSpec((bn, 128), lambda i: (i, 0))],
        out_shape=[jax.ShapeDtypeStruct((rows, 128), jnp.float32),
                   jax.ShapeDtypeStruct((rows, 128), jnp.float32)],
    )(x, pmat_l, pmat_r)


def _proj(W, a):
    ind = W.shape[0]
    heads, od = a.shape
    p = jnp.einsum("khd,hd->kh", W.reshape(ind, heads, od), a)
    return jnp.pad(p, ((0, 0), (0, 128 - heads)))


_MESH = dict(core_axis_name="c", subcore_axis_name="s")

_GDN = lax.GatherDimensionNumbers(
    offset_dims=(), collapsed_slice_dims=(0,), start_index_map=(0,))


def _lane_gather(vec, idxvec):
    return lax.gather(vec, idxvec[:, None], _GDN, (1,),
                      mode=lax.GatherScatterMode.PROMISE_IN_BOUNDS)


def _edge_softmax(elv, erv, src, dst, half, e):
    nbe = 80
    et = e // 16
    nb = et // nbe
    pd = half // 8 + 8
    zblocks = [(o, min(128, pd - o)) for o in range(0, pd, 128)]
    src3 = src.reshape(16, nb, nbe)
    dst3 = dst.reshape(16, nb, nbe)

    @functools.partial(
        pl.kernel, mesh=plsc.VectorSubcoreMesh(**_MESH),
        out_type=[jax.ShapeDtypeStruct((16 * nb, nbe, 16), jnp.float32),
                  jax.ShapeDtypeStruct((2 * pd, 128), jnp.float32)],
        scratch_types=[
            pltpu.VMEM((nb, nbe), jnp.int32),
            pltpu.VMEM((nb, nbe), jnp.int32),
            pltpu.VMEM((1, nbe), jnp.int32),
            pltpu.VMEM((nbe, 128), jnp.float32),
            pltpu.VMEM((nbe, 128), jnp.float32),
            pltpu.VMEM((nbe, 128), jnp.float32),
            pltpu.VMEM((nbe, 16), jnp.float32),
            pltpu.VMEM((128, 128), jnp.float32),
            pltpu.VMEM_SHARED((pd, 128), jnp.float32),
        ],
    )
    def a1(elvg, ervg, srcg, dstg, ee_out, den_out,
           src_v, dst_v, idxb, elb, erb, eeb, eec, zfb, dacc):
        c = lax.axis_index("c")
        s = lax.axis_index("s")
        base = c * half
        pltpu.sync_copy(srcg.at[s], src_v)
        pltpu.sync_copy(dstg.at[s], dst_v)

        def zrow(i, _):
            for v in range(8):
                zfb[i, pl.ds(16 * v, 16)] = jnp.zeros((16,), jnp.float32)
            return 0
        lax.fori_loop(0, 128, zrow, 0)

        @pl.when(s == 0)
        def _():
            for bo, cnt in zblocks:
                pltpu.sync_copy(zfb.at[pl.ds(0, cnt)],
                                dacc.at[pl.ds(bo, cnt)])
        plsc.subcore_barrier()

        def batch(b, _):
            pltpu.sync_copy(elvg.at[src_v.at[b]], elb)
            pltpu.sync_copy(ervg.at[dst_v.at[b]], erb)
            for v in range(nbe // 16):
                d = dst_v[b, pl.ds(16 * v, 16)] - base
                inb = jnp.logical_and(d >= 0, d < half)
                loc = jnp.where(inb, d, half)
                idxb[0, pl.ds(16 * v, 16)] = lax.shift_right_logical(loc, 3)
                slotf = jnp.bitwise_and(loc, 7).astype(jnp.float32)

                def edge(j, _):
                    jj = v * 16 + j
                    ev = elb[jj, pl.ds(0, 16)] + erb[jj, pl.ds(0, 16)]
                    ev = jnp.maximum(ev, 0.0) + 0.2 * jnp.minimum(ev, 0.0)
                    w = jnp.exp(ev)
                    eec[jj] = w
                    sv = _lane_gather(slotf, jnp.full((16,), j, jnp.int32))
                    for u in range(8):
                        d2 = (sv - float(u)) * (sv - float(u))
                        ind = jnp.maximum(1.0 - d2, 0.0)
                        eeb[jj, pl.ds(16 * u, 16)] = w * ind
                    return 0
                lax.fori_loop(0, 16, edge, 0)

            @pl.when(c == 0)
            def _():
                pltpu.sync_copy(eec, ee_out.at[s * nb + b])
            pltpu.sync_copy(eeb, dacc.at[idxb.at[0]], add=True)
            return 0
        lax.fori_loop(0, nb, batch, 0)
        plsc.subcore_barrier()

        @pl.when(s < (half // 8) // 128)
        def _():
            pltpu.sync_copy(dacc.at[pl.ds(s * 128, 128)], zfb)
            pltpu.sync_copy(zfb, den_out.at[pl.ds(c * pd + s * 128, 128)])

    return a1(elv, erv, src3, dst3)


def _aggregate(featT, ee, denp, src, dst, bias, half, e, nchunk, hid_chunks,
               do_relu):
    nbe = 80
    et = e // 16
    nb = et // nbe
    tpz = half // 16
    zblocks = [(o, min(128, tpz - o)) for o in range(0, tpz, 128)]
    nunit = half // 128
    pd = half // 8 + 8
    ftab_rows = featT.shape[1]
    featflat = featT.reshape(nchunk * ftab_rows, 128)
    src3 = src.reshape(16, nb, nbe)
    dst3 = dst.reshape(16, nb, nbe)
    al3 = ee.reshape(16 * nb, nbe, 16)

    @functools.partial(
        pl.kernel, mesh=plsc.VectorSubcoreMesh(**_MESH),
        out_type=jax.ShapeDtypeStruct((nchunk * 2 * half, 128), jnp.float32),
        scratch_types=[
            pltpu.VMEM((nb, nbe), jnp.int32),
            pltpu.VMEM((nb, nbe), jnp.int32),
            pltpu.VMEM((nbe,), jnp.int32),
            pltpu.VMEM((1, nbe), jnp.int32),
            pltpu.VMEM((nbe, 16), jnp.float32),
            pltpu.VMEM((nbe, 128), jnp.float32),
            pltpu.VMEM((nchunk, 128), jnp.float32),
            pltpu.VMEM((128, 128), jnp.float32),
            pltpu.VMEM((128, 128), jnp.float32),
            pltpu.VMEM((16, 128), jnp.float32),
            pltpu.VMEM_SHARED((half + 8, 128), jnp.float32),
        ],
    )
    def bk(featg, alg, deng, srcg, dstg, biasg, hout,
           src_v, dst_v, idx2, idxb, alb, msgb, bias_v, zbuf, fbuf, d0b, acc):
        c = lax.axis_index("c")
        s = lax.axis_index("s")
        base = c * half
        pltpu.sync_copy(srcg.at[s], src_v)
        pltpu.sync_copy(dstg.at[s], dst_v)
        pltpu.sync_copy(biasg, bias_v)

        def zrow(i, _):
            for v in range(8):
                zbuf[i, pl.ds(16 * v, 16)] = jnp.zeros((16,), jnp.float32)
            return 0
        lax.fori_loop(0, 128, zrow, 0)

        def chunk_body(fc, _):
            h = fc // hid_chunks
            hvec = jnp.full((16,), h, dtype=jnp.int32)
            off = fc * ftab_rows

            for bo, cnt in zblocks:
                pltpu.sync_copy(zbuf.at[pl.ds(0, cnt)],
                                acc.at[pl.ds(s * tpz + bo, cnt)])
            plsc.subcore_barrier()

            def batch(b, _):
                for v in range(nbe // 16):
                    idx2[pl.ds(16 * v, 16)] = src_v[b, pl.ds(16 * v, 16)] + off
                    d = dst_v[b, pl.ds(16 * v, 16)] - base
                    inb = jnp.logical_and(d >= 0, d < half)
                    idxb[0, pl.ds(16 * v, 16)] = jnp.where(inb, d, half)
                pltpu.sync_copy(featg.at[idx2], msgb)
                pltpu.sync_copy(alg.at[s * nb + b], alb)

                def edge(j, _):
                    sp = _lane_gather(alb[j], hvec)
                    for v in range(8):
                        msgb[j, pl.ds(16 * v, 16)] = msgb[j, pl.ds(16 * v, 16)] * sp
                    return 0
                lax.fori_loop(0, nbe, edge, 0)
                pltpu.sync_copy(msgb, acc.at[idxb.at[0]], add=True)
                return 0
            lax.fori_loop(0, nb, batch, 0)
            plsc.subcore_barrier()

            bvr = [bias_v[fc, pl.ds(16 * v, 16)] for v in range(8)]
            for ui in range((nunit + 15) // 16):
                un = s + 16 * ui

                @pl.when(un < nunit)
                def _():
                    r0 = un * 128
                    pltpu.sync_copy(acc.at[pl.ds(r0, 128)], fbuf)
                    pltpu.sync_copy(deng.at[pl.ds(c * pd + un * 16, 16)], d0b)

                    def frow(ip, _):
                        for u in range(8):
                            dv = d0b[ip, pl.ds(16 * u, 16)]
                            rec = 1.0 / jnp.maximum(_lane_gather(dv, hvec),
                                                    1e-30)
                            for v in range(8):
                                x = fbuf[ip * 8 + u, pl.ds(16 * v, 16)] * rec \
                                    + bvr[v]
                                if do_relu:
                                    x = jnp.maximum(x, 0.0)
                                fbuf[ip * 8 + u, pl.ds(16 * v, 16)] = x
                        return 0
                    lax.fori_loop(0, 16, frow, 0)
                    pltpu.sync_copy(
                        fbuf, hout.at[pl.ds(fc * 2 * half + base + r0, 128)])
            plsc.subcore_barrier()
            return 0

        lax.fori_loop(0, nchunk, chunk_body, 0)

    out = bk(featflat, al3, denp, src3, dst3, bias)
    return out.reshape(nchunk, 2 * half, 128)


def _token_gather(h3, gid):
    kc, rows, _ = h3.shape
    hflat = h3.reshape(kc * rows, 128)
    t = gid.shape[0]
    tt = t // 32

    @functools.partial(
        pl.kernel, mesh=plsc.VectorSubcoreMesh(**_MESH),
        out_type=jax.ShapeDtypeStruct((kc * t, 128), jnp.float32),
        scratch_types=[
            pltpu.VMEM((tt,), jnp.int32),
            pltpu.VMEM((tt,), jnp.int32),
            pltpu.VMEM((tt, 128), jnp.float32),
        ],
    )
    def tk(hg, gg, out, gid_v, idx2, rowb):
        c = lax.axis_index("c")
        s = lax.axis_index("s")
        wid = c * 16 + s
        pltpu.sync_copy(gg.at[pl.ds(wid * tt, tt)], gid_v)
        for ci in range(kc):
            for v in range(tt // 16):
                idx2[pl.ds(16 * v, 16)] = gid_v[pl.ds(16 * v, 16)] + ci * rows
            pltpu.sync_copy(hg.at[idx2], rowb)
            pltpu.sync_copy(rowb, out.at[pl.ds(ci * t + wid * tt, tt)])

    return tk(hflat, gid).reshape(kc, t, 128)


def _gat_layer(x, src, dst, W, al, ar, b, half, e, bn, do_relu):
    heads, od = al.shape
    featT = _featT(x, W, bn=bn)
    elv, erv = _elr(x, _proj(W, al), _proj(W, ar), bn=bn)
    ee, denp = _edge_softmax(elv, erv, src, dst, half, e)
    nchunk = (heads * od) // 128
    bias = b.reshape(nchunk, 128)
    return _aggregate(featT, ee, denp, src, dst, bias, half, e,
                      nchunk, od // 128, do_relu)


def kernel(instance_embs_batch, edge_index, instance_batch_local_token_ids,
           node_counts, W1, al1, ar1, b1, W2, al2, ar2, b2):
    x = instance_embs_batch
    n = x.shape[0]
    e = edge_index.shape[1]
    half = 1024 * ((n + 2047) // 2048)
    src = edge_index[0]
    dst = edge_index[1]

    h = _gat_layer(x, src, dst, W1, al1, ar1, b1, half, e,
                   bn=n // 10, do_relu=True)
    h = _gat_layer(h, src, dst, W2, al2, ar2, b2, half, e,
                   bn=2 * half // 8, do_relu=False)

    bsz, lsz = instance_batch_local_token_ids.shape
    offsets = jnp.concatenate([jnp.zeros((1,), dtype=node_counts.dtype),
                               jnp.cumsum(node_counts)[:-1]])
    gid = (instance_batch_local_token_ids + offsets[:, None]).reshape(bsz * lsz)
    pc = _token_gather(h, gid)
    preds = jnp.transpose(pc, (1, 0, 2)).reshape(bsz, lsz, al2.shape[1])
    return jnp.squeeze(preds)


# speedup vs baseline: 5.9167x; 1.2038x over previous
import functools

import jax
import jax.numpy as jnp
from jax import lax
from jax.experimental import pallas as pl
from jax.experimental.pallas import tpu as pltpu
from jax.experimental.pallas import tpu_sc as plsc


def _xin(x):
    if x.ndim == 2:
        rows, k = x.shape
        return rows, k, x, lambda bn: pl.BlockSpec((bn, k), lambda i: (i, 0)), \
            lambda r: r[...]
    kc, rows, _ = x.shape
    return rows, kc * 128, x, \
        lambda bn: pl.BlockSpec((kc, bn, 128), lambda i: (0, i, 0)), \
        lambda r: jnp.concatenate([r[i] for i in range(kc)], axis=-1)


def _featT_body(nchunk, get, x_ref, w_ref, o_ref):
    f = jnp.dot(get(x_ref), w_ref[...], preferred_element_type=jnp.float32)
    for c in range(nchunk):
        o_ref[c] = f[:, c * 128:(c + 1) * 128]


def _featT(x, w, bn=1000):
    rows, k, x, xspec, get = _xin(x)
    m = w.shape[1]
    nchunk = m // 128
    return pl.pallas_call(
        functools.partial(_featT_body, nchunk, get),
        grid=(rows // bn,),
        in_specs=[xspec(bn), pl.BlockSpec((k, m), lambda i: (0, 0))],
        out_specs=pl.BlockSpec((nchunk, bn, 128), lambda i: (0, i, 0)),
        out_shape=jax.ShapeDtypeStruct((nchunk, rows, 128), jnp.float32),
    )(x, w)


def _elr_body(get, x_ref, pl_ref, pr_ref, el_ref, er_ref):
    xv = get(x_ref)
    el_ref[...] = jnp.dot(xv, pl_ref[...], preferred_element_type=jnp.float32)
    er_ref[...] = jnp.dot(xv, pr_ref[...], preferred_element_type=jnp.float32)


def _elr(x, pmat_l, pmat_r, bn=1000):
    rows, k, x, xspec, get = _xin(x)
    return pl.pallas_call(
        functools.partial(_elr_body, get),
        grid=(rows // bn,),
        in_specs=[xspec(bn),
                  pl.BlockSpec((k, 128), lambda i: (0, 0)),
                  pl.BlockSpec((k, 128), lambda i: (0, 0))],
        out_specs=[pl.BlockSpec((bn, 128), lambda i: (i, 0)),
                   pl.BlockSpec((bn, 128), lambda i: (i, 0))],
        out_shape=[jax.ShapeDtypeStruct((rows, 128), jnp.float32),
                   jax.ShapeDtypeStruct((rows, 128), jnp.float32)],
    )(x, pmat_l, pmat_r)


def _proj(W, a):
    ind = W.shape[0]
    heads, od = a.shape
    p = jnp.einsum("khd,hd->kh", W.reshape(ind, heads, od), a)
    return jnp.pad(p, ((0, 0), (0, 128 - heads)))


_MESH = dict(core_axis_name="c", subcore_axis_name="s")

_GDN = lax.GatherDimensionNumbers(
    offset_dims=(), collapsed_slice_dims=(0,), start_index_map=(0,))


def _lane_gather(vec, idxvec):
    return lax.gather(vec, idxvec[:, None], _GDN, (1,),
                      mode=lax.GatherScatterMode.PROMISE_IN_BOUNDS)


def _edge_softmax(elv, erv, src, dst, half, e):
    nbe = 80
    et = e // 16
    nb = et // nbe
    pd = half // 8 + 8
    zblocks = [(o, min(128, pd - o)) for o in range(0, pd, 128)]
    src3 = src.reshape(16, nb, nbe)
    dst3 = dst.reshape(16, nb, nbe)

    @functools.partial(
        pl.kernel, mesh=plsc.VectorSubcoreMesh(**_MESH),
        out_type=[jax.ShapeDtypeStruct((16 * nb, nbe, 16), jnp.float32),
                  jax.ShapeDtypeStruct((2 * pd, 128), jnp.float32)],
        scratch_types=[
            pltpu.VMEM((nb, nbe), jnp.int32),
            pltpu.VMEM((nb, nbe), jnp.int32),
            pltpu.VMEM((1, nbe), jnp.int32),
            pltpu.VMEM((nbe, 128), jnp.float32),
            pltpu.VMEM((nbe, 128), jnp.float32),
            pltpu.VMEM((nbe, 128), jnp.float32),
            pltpu.VMEM((nbe, 16), jnp.float32),
            pltpu.VMEM((128, 128), jnp.float32),
            pltpu.VMEM_SHARED((pd, 128), jnp.float32),
        ],
    )
    def a1(elvg, ervg, srcg, dstg, ee_out, den_out,
           src_v, dst_v, idxb, elb, erb, eeb, eec, zfb, dacc):
        c = lax.axis_index("c")
        s = lax.axis_index("s")
        base = c * half
        pltpu.sync_copy(srcg.at[s], src_v)
        pltpu.sync_copy(dstg.at[s], dst_v)

        def zrow(i, _):
            for v in range(8):
                zfb[i, pl.ds(16 * v, 16)] = jnp.zeros((16,), jnp.float32)
            return 0
        lax.fori_loop(0, 128, zrow, 0)

        @pl.when(s == 0)
        def _():
            for bo, cnt in zblocks:
                pltpu.sync_copy(zfb.at[pl.ds(0, cnt)],
                                dacc.at[pl.ds(bo, cnt)])
        plsc.subcore_barrier()

        def batch(b, _):
            pltpu.sync_copy(elvg.at[src_v.at[b]], elb)
            pltpu.sync_copy(ervg.at[dst_v.at[b]], erb)
            for v in range(nbe // 16):
                d = dst_v[b, pl.ds(16 * v, 16)] - base
                inb = jnp.logical_and(d >= 0, d < half)
                loc = jnp.where(inb, d, half)
                idxb[0, pl.ds(16 * v, 16)] = lax.shift_right_logical(loc, 3)
                slotf = jnp.bitwise_and(loc, 7).astype(jnp.float32)

                def edge(j, _):
                    jj = v * 16 + j
                    ev = elb[jj, pl.ds(0, 16)] + erb[jj, pl.ds(0, 16)]
                    ev = jnp.maximum(ev, 0.0) + 0.2 * jnp.minimum(ev, 0.0)
                    w = jnp.exp(ev)
                    eec[jj] = w
                    sv = _lane_gather(slotf, jnp.full((16,), j, jnp.int32))
                    for u in range(8):
                        d2 = (sv - float(u)) * (sv - float(u))
                        ind = jnp.maximum(1.0 - d2, 0.0)
                        eeb[jj, pl.ds(16 * u, 16)] = w * ind
                    return 0
                lax.fori_loop(0, 16, edge, 0)

            @pl.when(c == 0)
            def _():
                pltpu.sync_copy(eec, ee_out.at[s * nb + b])
            pltpu.sync_copy(eeb, dacc.at[idxb.at[0]], add=True)
            return 0
        lax.fori_loop(0, nb, batch, 0)
        plsc.subcore_barrier()

        @pl.when(s < (half // 8) // 128)
        def _():
            pltpu.sync_copy(dacc.at[pl.ds(s * 128, 128)], zfb)
            pltpu.sync_copy(zfb, den_out.at[pl.ds(c * pd + s * 128, 128)])

    return a1(elv, erv, src3, dst3)


def _aggregate(featT, ee, denp, src, dst, bias, half, e, nchunk, hid_chunks,
               do_relu):
    nbe = 80
    et = e // 16
    nb = et // nbe
    tpz = half // 16
    zblocks = [(o, min(128, tpz - o)) for o in range(0, tpz, 128)]
    nunit = half // 128
    pd = half // 8 + 8
    ftab_rows = featT.shape[1]
    featflat = featT.reshape(nchunk * ftab_rows, 128)
    src3 = src.reshape(16, nb, nbe)
    dst3 = dst.reshape(16, nb, nbe)
    al3 = ee.reshape(16 * nb, nbe, 16)

    @functools.partial(
        pl.kernel, mesh=plsc.VectorSubcoreMesh(**_MESH),
        out_type=jax.ShapeDtypeStruct((nchunk * 2 * half, 128), jnp.float32),
        scratch_types=[
            pltpu.VMEM((nb, nbe), jnp.int32),
            pltpu.VMEM((nb, nbe), jnp.int32),
            pltpu.VMEM((2, nbe), jnp.int32),
            pltpu.VMEM((1, nbe), jnp.int32),
            pltpu.VMEM((nbe, 16), jnp.float32),
            pltpu.VMEM((nbe, 128), jnp.float32),
            pltpu.VMEM((nbe, 128), jnp.float32),
            pltpu.VMEM((nchunk, 128), jnp.float32),
            pltpu.VMEM((128, 128), jnp.float32),
            pltpu.VMEM((128, 128), jnp.float32),
            pltpu.VMEM((16, 128), jnp.float32),
            pltpu.SemaphoreType.DMA,
            pltpu.SemaphoreType.DMA,
            pltpu.VMEM_SHARED((half + 8, 128), jnp.float32),
        ],
    )
    def bk(featg, alg, deng, srcg, dstg, biasg, hout,
           src_v, dst_v, idxA, idxb, alb, msg0, msg1, bias_v, zbuf, fbuf, d0b,
           sem0, sem1, acc):
        c = lax.axis_index("c")
        s = lax.axis_index("s")
        base = c * half
        pltpu.sync_copy(srcg.at[s], src_v)
        pltpu.sync_copy(dstg.at[s], dst_v)
        pltpu.sync_copy(biasg, bias_v)

        def zrow(i, _):
            for v in range(8):
                zbuf[i, pl.ds(16 * v, 16)] = jnp.zeros((16,), jnp.float32)
            return 0
        lax.fori_loop(0, 128, zrow, 0)

        def chunk_body(fc, _):
            h = fc // hid_chunks
            hvec = jnp.full((16,), h, dtype=jnp.int32)
            off = fc * ftab_rows

            for bo, cnt in zblocks:
                pltpu.sync_copy(zbuf.at[pl.ds(0, cnt)],
                                acc.at[pl.ds(s * tpz + bo, cnt)])
            plsc.subcore_barrier()

            def batch(b, _):
                for v in range(nbe // 16):
                    idxA[0, pl.ds(16 * v, 16)] = (
                        src_v[b, pl.ds(16 * v, 16)] + off)
                    d = dst_v[b, pl.ds(16 * v, 16)] - base
                    inb = jnp.logical_and(d >= 0, d < half)
                    idxb[0, pl.ds(16 * v, 16)] = jnp.where(inb, d, half)
                pltpu.async_copy(alg.at[s * nb + b], alb, sem1)
                pltpu.async_copy(featg.at[idxA.at[0]], msg0, sem0)
                pltpu.make_async_copy(alg.at[s * nb + b], alb, sem1).wait()
                pltpu.make_async_copy(featg.at[idxA.at[0]], msg0, sem0).wait()

                def edge(j, _):
                    sp = _lane_gather(alb[j], hvec)
                    for v in range(8):
                        msg0[j, pl.ds(16 * v, 16)] = msg0[j, pl.ds(16 * v, 16)] * sp
                    return 0
                lax.fori_loop(0, nbe, edge, 0)
                pltpu.sync_copy(msg0, acc.at[idxb.at[0]], add=True)
                return 0
            lax.fori_loop(0, nb, batch, 0)
            plsc.subcore_barrier()

            bvr = [bias_v[fc, pl.ds(16 * v, 16)] for v in range(8)]
            for ui in range((nunit + 15) // 16):
                un = s + 16 * ui

                @pl.when(un < nunit)
                def _():
                    r0 = un * 128
                    pltpu.sync_copy(acc.at[pl.ds(r0, 128)], fbuf)
                    pltpu.sync_copy(deng.at[pl.ds(c * pd + un * 16, 16)], d0b)

                    def frow(ip, _):
                        for u in range(8):
                            dv = d0b[ip, pl.ds(16 * u, 16)]
                            rec = 1.0 / jnp.maximum(_lane_gather(dv, hvec),
                                                    1e-30)
                            for v in range(8):
                                x = fbuf[ip * 8 + u, pl.ds(16 * v, 16)] * rec \
                                    + bvr[v]
                                if do_relu:
                                    x = jnp.maximum(x, 0.0)
                                fbuf[ip * 8 + u, pl.ds(16 * v, 16)] = x
                        return 0
                    lax.fori_loop(0, 16, frow, 0)
                    pltpu.sync_copy(
                        fbuf, hout.at[pl.ds(fc * 2 * half + base + r0, 128)])
            plsc.subcore_barrier()
            return 0

        lax.fori_loop(0, nchunk, chunk_body, 0)

    out = bk(featflat, al3, denp, src3, dst3, bias)
    return out.reshape(nchunk, 2 * half, 128)


def _token_gather(h3, gid):
    kc, rows, _ = h3.shape
    hflat = h3.reshape(kc * rows, 128)
    t = gid.shape[0]
    tt = t // 32

    @functools.partial(
        pl.kernel, mesh=plsc.VectorSubcoreMesh(**_MESH),
        out_type=jax.ShapeDtypeStruct((kc * t, 128), jnp.float32),
        scratch_types=[
            pltpu.VMEM((tt,), jnp.int32),
            pltpu.VMEM((tt,), jnp.int32),
            pltpu.VMEM((tt, 128), jnp.float32),
        ],
    )
    def tk(hg, gg, out, gid_v, idx2, rowb):
        c = lax.axis_index("c")
        s = lax.axis_index("s")
        wid = c * 16 + s
        pltpu.sync_copy(gg.at[pl.ds(wid * tt, tt)], gid_v)
        for ci in range(kc):
            for v in range(tt // 16):
                idx2[pl.ds(16 * v, 16)] = gid_v[pl.ds(16 * v, 16)] + ci * rows
            pltpu.sync_copy(hg.at[idx2], rowb)
            pltpu.sync_copy(rowb, out.at[pl.ds(ci * t + wid * tt, tt)])

    return tk(hflat, gid).reshape(kc, t, 128)


def _gat_layer(x, src, dst, W, al, ar, b, half, e, bn, do_relu):
    heads, od = al.shape
    featT = _featT(x, W, bn=bn)
    elv, erv = _elr(x, _proj(W, al), _proj(W, ar), bn=bn)
    ee, denp = _edge_softmax(elv, erv, src, dst, half, e)
    nchunk = (heads * od) // 128
    bias = b.reshape(nchunk, 128)
    return _aggregate(featT, ee, denp, src, dst, bias, half, e,
                      nchunk, od // 128, do_relu)


def kernel(instance_embs_batch, edge_index, instance_batch_local_token_ids,
           node_counts, W1, al1, ar1, b1, W2, al2, ar2, b2):
    x = instance_embs_batch
    n = x.shape[0]
    e = edge_index.shape[1]
    half = 1024 * ((n + 2047) // 2048)
    src = edge_index[0]
    dst = edge_index[1]

    h = _gat_layer(x, src, dst, W1, al1, ar1, b1, half, e,
                   bn=n // 10, do_relu=True)
    h = _gat_layer(h, src, dst, W2, al2, ar2, b2, half, e,
                   bn=2 * half // 8, do_relu=False)

    bsz, lsz = instance_batch_local_token_ids.shape
    offsets = jnp.concatenate([jnp.zeros((1,), dtype=node_counts.dtype),
                               jnp.cumsum(node_counts)[:-1]])
    gid = (instance_batch_local_token_ids + offsets[:, None]).reshape(bsz * lsz)
    pc = _token_gather(h, gid)
    preds = jnp.transpose(pc, (1, 0, 2)).reshape(bsz, lsz, al2.shape[1])
    return jnp.squeeze(preds)


# speedup vs baseline: 6.1773x; 1.0440x over previous
import functools

import jax
import jax.numpy as jnp
from jax import lax
from jax.experimental import pallas as pl
from jax.experimental.pallas import tpu as pltpu
from jax.experimental.pallas import tpu_sc as plsc


def _xin(x):
    if x.ndim == 2:
        rows, k = x.shape
        return rows, k, x, lambda bn: pl.BlockSpec((bn, k), lambda i: (i, 0)), \
            lambda r: r[...]
    kc, rows, _ = x.shape
    return rows, kc * 128, x, \
        lambda bn: pl.BlockSpec((kc, bn, 128), lambda i: (0, i, 0)), \
        lambda r: jnp.concatenate([r[i] for i in range(kc)], axis=-1)


def _featT_body(nchunk, get, x_ref, w_ref, o_ref):
    f = jnp.dot(get(x_ref), w_ref[...], preferred_element_type=jnp.float32)
    for c in range(nchunk):
        o_ref[c] = f[:, c * 128:(c + 1) * 128]


def _featT(x, w, bn=1000):
    rows, k, x, xspec, get = _xin(x)
    m = w.shape[1]
    nchunk = m // 128
    return pl.pallas_call(
        functools.partial(_featT_body, nchunk, get),
        grid=(rows // bn,),
        in_specs=[xspec(bn), pl.BlockSpec((k, m), lambda i: (0, 0))],
        out_specs=pl.BlockSpec((nchunk, bn, 128), lambda i: (0, i, 0)),
        out_shape=jax.ShapeDtypeStruct((nchunk, rows, 128), jnp.float32),
    )(x, w)


def _elr_body(get, x_ref, pl_ref, pr_ref, el_ref, er_ref):
    xv = get(x_ref)
    el_ref[...] = jnp.dot(xv, pl_ref[...], preferred_element_type=jnp.float32)
    er_ref[...] = jnp.dot(xv, pr_ref[...], preferred_element_type=jnp.float32)


def _elr(x, pmat_l, pmat_r, bn=1000):
    rows, k, x, xspec, get = _xin(x)
    return pl.pallas_call(
        functools.partial(_elr_body, get),
        grid=(rows // bn,),
        in_specs=[xspec(bn),
                  pl.BlockSpec((k, 128), lambda i: (0, 0)),
                  pl.BlockSpec((k, 128), lambda i: (0, 0))],
        out_specs=[pl.BlockSpec((bn, 128), lambda i: (i, 0)),
                   pl.BlockSpec((bn, 128), lambda i: (i, 0))],
        out_shape=[jax.ShapeDtypeStruct((rows, 128), jnp.float32),
                   jax.ShapeDtypeStruct((rows, 128), jnp.float32)],
    )(x, pmat_l, pmat_r)


def _proj(W, a):
    ind = W.shape[0]
    heads, od = a.shape
    p = jnp.einsum("khd,hd->kh", W.reshape(ind, heads, od), a)
    return jnp.pad(p, ((0, 0), (0, 128 - heads)))


_MESH = dict(core_axis_name="c", subcore_axis_name="s")

_GDN = lax.GatherDimensionNumbers(
    offset_dims=(), collapsed_slice_dims=(0,), start_index_map=(0,))


def _lane_gather(vec, idxvec):
    return lax.gather(vec, idxvec[:, None], _GDN, (1,),
                      mode=lax.GatherScatterMode.PROMISE_IN_BOUNDS)


def _edge_softmax(elv, erv, src, dst, half, e):
    nbe = 80
    et = e // 16
    nb = et // nbe
    pd = half // 8 + 8
    zblocks = [(o, min(128, pd - o)) for o in range(0, pd, 128)]
    src3 = src.reshape(16, nb, nbe)
    dst3 = dst.reshape(16, nb, nbe)

    @functools.partial(
        pl.kernel, mesh=plsc.VectorSubcoreMesh(**_MESH),
        out_type=[jax.ShapeDtypeStruct((16 * nb, nbe, 16), jnp.float32),
                  jax.ShapeDtypeStruct((2 * pd, 128), jnp.float32)],
        scratch_types=[
            pltpu.VMEM((nb, nbe), jnp.int32),
            pltpu.VMEM((nb, nbe), jnp.int32),
            pltpu.VMEM((1, nbe), jnp.int32),
            pltpu.VMEM((nbe, 128), jnp.float32),
            pltpu.VMEM((nbe, 128), jnp.float32),
            pltpu.VMEM((nbe, 128), jnp.float32),
            pltpu.VMEM((nbe, 16), jnp.float32),
            pltpu.VMEM((128, 128), jnp.float32),
            pltpu.SemaphoreType.DMA,
            pltpu.SemaphoreType.DMA,
            pltpu.VMEM_SHARED((pd, 128), jnp.float32),
        ],
    )
    def a1(elvg, ervg, srcg, dstg, ee_out, den_out,
           src_v, dst_v, idxb, elb, erb, eeb, eec, zfb, sema, semb, dacc):
        c = lax.axis_index("c")
        s = lax.axis_index("s")
        base = c * half
        pltpu.sync_copy(srcg.at[s], src_v)
        pltpu.sync_copy(dstg.at[s], dst_v)

        def zrow(i, _):
            for v in range(8):
                zfb[i, pl.ds(16 * v, 16)] = jnp.zeros((16,), jnp.float32)
            return 0
        lax.fori_loop(0, 128, zrow, 0)

        @pl.when(s == 0)
        def _():
            for bo, cnt in zblocks:
                pltpu.sync_copy(zfb.at[pl.ds(0, cnt)],
                                dacc.at[pl.ds(bo, cnt)])
        plsc.subcore_barrier()

        def batch(b, _):
            pltpu.async_copy(elvg.at[src_v.at[b]], elb, sema)
            pltpu.async_copy(ervg.at[dst_v.at[b]], erb, semb)
            pltpu.make_async_copy(elvg.at[src_v.at[b]], elb, sema).wait()
            pltpu.make_async_copy(ervg.at[dst_v.at[b]], erb, semb).wait()
            for v in range(nbe // 16):
                d = dst_v[b, pl.ds(16 * v, 16)] - base
                inb = jnp.logical_and(d >= 0, d < half)
                loc = jnp.where(inb, d, half)
                idxb[0, pl.ds(16 * v, 16)] = lax.shift_right_logical(loc, 3)
                slotf = jnp.bitwise_and(loc, 7).astype(jnp.float32)

                def edge(j, _):
                    jj = v * 16 + j
                    ev = elb[jj, pl.ds(0, 16)] + erb[jj, pl.ds(0, 16)]
                    ev = jnp.maximum(ev, 0.0) + 0.2 * jnp.minimum(ev, 0.0)
                    w = jnp.exp(ev)
                    eec[jj] = w
                    sv = _lane_gather(slotf, jnp.full((16,), j, jnp.int32))
                    for u in range(8):
                        d2 = (sv - float(u)) * (sv - float(u))
                        ind = jnp.maximum(1.0 - d2, 0.0)
                        eeb[jj, pl.ds(16 * u, 16)] = w * ind
                    return 0
                lax.fori_loop(0, 16, edge, 0)

            @pl.when(c == 0)
            def _():
                pltpu.sync_copy(eec, ee_out.at[s * nb + b])
            pltpu.sync_copy(eeb, dacc.at[idxb.at[0]], add=True)
            return 0
        lax.fori_loop(0, nb, batch, 0)
        plsc.subcore_barrier()

        @pl.when(s < (half // 8) // 128)
        def _():
            pltpu.sync_copy(dacc.at[pl.ds(s * 128, 128)], zfb)
            pltpu.sync_copy(zfb, den_out.at[pl.ds(c * pd + s * 128, 128)])

    return a1(elv, erv, src3, dst3)


def _aggregate(featT, ee, denp, src, dst, bias, half, e, nchunk, hid_chunks,
               do_relu):
    nbe = 80
    et = e // 16
    nb = et // nbe
    tpz = half // 16
    zblocks = [(o, min(128, tpz - o)) for o in range(0, tpz, 128)]
    nunit = half // 128
    pd = half // 8 + 8
    ftab_rows = featT.shape[1]
    featflat = featT.reshape(nchunk * ftab_rows, 128)
    src3 = src.reshape(16, nb, nbe)
    dst3 = dst.reshape(16, nb, nbe)
    al3 = ee.reshape(16 * nb, nbe, 16)

    @functools.partial(
        pl.kernel, mesh=plsc.VectorSubcoreMesh(**_MESH),
        out_type=jax.ShapeDtypeStruct((nchunk * 2 * half, 128), jnp.float32),
        scratch_types=[
            pltpu.VMEM((nb, nbe), jnp.int32),
            pltpu.VMEM((nb, nbe), jnp.int32),
            pltpu.VMEM((2, nbe), jnp.int32),
            pltpu.VMEM((1, nbe), jnp.int32),
            pltpu.VMEM((nbe, 16), jnp.float32),
            pltpu.VMEM((nbe, 128), jnp.float32),
            pltpu.VMEM((nbe, 128), jnp.float32),
            pltpu.VMEM((nchunk, 128), jnp.float32),
            pltpu.VMEM((128, 128), jnp.float32),
            pltpu.VMEM((128, 128), jnp.float32),
            pltpu.VMEM((16, 128), jnp.float32),
            pltpu.SemaphoreType.DMA,
            pltpu.SemaphoreType.DMA,
            pltpu.VMEM_SHARED((half + 8, 128), jnp.float32),
        ],
    )
    def bk(featg, alg, deng, srcg, dstg, biasg, hout,
           src_v, dst_v, idxA, idxb, alb, msg0, msg1, bias_v, zbuf, fbuf, d0b,
           sem0, sem1, acc):
        c = lax.axis_index("c")
        s = lax.axis_index("s")
        base = c * half
        pltpu.sync_copy(srcg.at[s], src_v)
        pltpu.sync_copy(dstg.at[s], dst_v)
        pltpu.sync_copy(biasg, bias_v)

        def zrow(i, _):
            for v in range(8):
                zbuf[i, pl.ds(16 * v, 16)] = jnp.zeros((16,), jnp.float32)
            return 0
        lax.fori_loop(0, 128, zrow, 0)

        def chunk_body(fc, _):
            h = fc // hid_chunks
            hvec = jnp.full((16,), h, dtype=jnp.int32)
            off = fc * ftab_rows

            for bo, cnt in zblocks:
                pltpu.sync_copy(zbuf.at[pl.ds(0, cnt)],
                                acc.at[pl.ds(s * tpz + bo, cnt)])
            plsc.subcore_barrier()

            def batch(b, _):
                for v in range(nbe // 16):
                    idxA[0, pl.ds(16 * v, 16)] = (
                        src_v[b, pl.ds(16 * v, 16)] + off)
                    d = dst_v[b, pl.ds(16 * v, 16)] - base
                    inb = jnp.logical_and(d >= 0, d < half)
                    idxb[0, pl.ds(16 * v, 16)] = jnp.where(inb, d, half)
                pltpu.async_copy(alg.at[s * nb + b], alb, sem1)
                pltpu.async_copy(featg.at[idxA.at[0]], msg0, sem0)
                pltpu.make_async_copy(alg.at[s * nb + b], alb, sem1).wait()
                pltpu.make_async_copy(featg.at[idxA.at[0]], msg0, sem0).wait()

                def edge(j, _):
                    sp = _lane_gather(alb[j], hvec)
                    for v in range(8):
                        msg0[j, pl.ds(16 * v, 16)] = msg0[j, pl.ds(16 * v, 16)] * sp
                    return 0
                lax.fori_loop(0, nbe, edge, 0)
                pltpu.sync_copy(msg0, acc.at[idxb.at[0]], add=True)
                return 0
            lax.fori_loop(0, nb, batch, 0)
            plsc.subcore_barrier()

            bvr = [bias_v[fc, pl.ds(16 * v, 16)] for v in range(8)]
            for ui in range((nunit + 15) // 16):
                un = s + 16 * ui

                @pl.when(un < nunit)
                def _():
                    r0 = un * 128
                    pltpu.sync_copy(acc.at[pl.ds(r0, 128)], fbuf)
                    pltpu.sync_copy(deng.at[pl.ds(c * pd + un * 16, 16)], d0b)

                    def frow(ip, _):
                        for u in range(8):
                            dv = d0b[ip, pl.ds(16 * u, 16)]
                            rec = 1.0 / jnp.maximum(_lane_gather(dv, hvec),
                                                    1e-30)
                            for v in range(8):
                                x = fbuf[ip * 8 + u, pl.ds(16 * v, 16)] * rec \
                                    + bvr[v]
                                if do_relu:
                                    x = jnp.maximum(x, 0.0)
                                fbuf[ip * 8 + u, pl.ds(16 * v, 16)] = x
                        return 0
                    lax.fori_loop(0, 16, frow, 0)
                    pltpu.sync_copy(
                        fbuf, hout.at[pl.ds(fc * 2 * half + base + r0, 128)])
            plsc.subcore_barrier()
            return 0

        lax.fori_loop(0, nchunk, chunk_body, 0)

    out = bk(featflat, al3, denp, src3, dst3, bias)
    return out.reshape(nchunk, 2 * half, 128)


def _token_gather(h3, gid):
    kc, rows, _ = h3.shape
    hflat = h3.reshape(kc * rows, 128)
    t = gid.shape[0]
    tt = t // 32

    @functools.partial(
        pl.kernel, mesh=plsc.VectorSubcoreMesh(**_MESH),
        out_type=jax.ShapeDtypeStruct((kc * t, 128), jnp.float32),
        scratch_types=[
            pltpu.VMEM((tt,), jnp.int32),
            pltpu.VMEM((tt,), jnp.int32),
            pltpu.VMEM((tt, 128), jnp.float32),
        ],
    )
    def tk(hg, gg, out, gid_v, idx2, rowb):
        c = lax.axis_index("c")
        s = lax.axis_index("s")
        wid = c * 16 + s
        pltpu.sync_copy(gg.at[pl.ds(wid * tt, tt)], gid_v)
        for ci in range(kc):
            for v in range(tt // 16):
                idx2[pl.ds(16 * v, 16)] = gid_v[pl.ds(16 * v, 16)] + ci * rows
            pltpu.sync_copy(hg.at[idx2], rowb)
            pltpu.sync_copy(rowb, out.at[pl.ds(ci * t + wid * tt, tt)])

    return tk(hflat, gid).reshape(kc, t, 128)


def _gat_layer(x, src, dst, W, al, ar, b, half, e, bn, do_relu):
    heads, od = al.shape
    featT = _featT(x, W, bn=bn)
    elv, erv = _elr(x, _proj(W, al), _proj(W, ar), bn=bn)
    ee, denp = _edge_softmax(elv, erv, src, dst, half, e)
    nchunk = (heads * od) // 128
    bias = b.reshape(nchunk, 128)
    return _aggregate(featT, ee, denp, src, dst, bias, half, e,
                      nchunk, od // 128, do_relu)


def kernel(instance_embs_batch, edge_index, instance_batch_local_token_ids,
           node_counts, W1, al1, ar1, b1, W2, al2, ar2, b2):
    x = instance_embs_batch
    n = x.shape[0]
    e = edge_index.shape[1]
    half = 1024 * ((n + 2047) // 2048)
    src = edge_index[0]
    dst = edge_index[1]

    h = _gat_layer(x, src, dst, W1, al1, ar1, b1, half, e,
                   bn=n // 10, do_relu=True)
    h = _gat_layer(h, src, dst, W2, al2, ar2, b2, half, e,
                   bn=2 * half // 8, do_relu=False)

    bsz, lsz = instance_batch_local_token_ids.shape
    offsets = jnp.concatenate([jnp.zeros((1,), dtype=node_counts.dtype),
                               jnp.cumsum(node_counts)[:-1]])
    gid = (instance_batch_local_token_ids + offsets[:, None]).reshape(bsz * lsz)
    pc = _token_gather(h, gid)
    preds = jnp.transpose(pc, (1, 0, 2)).reshape(bsz, lsz, al2.shape[1])
    return jnp.squeeze(preds)
